# baseline (speedup 1.0000x reference)
import jax
import jax.numpy as jnp
from jax import lax
from jax.experimental import pallas as pl
from jax.experimental.pallas import tpu as pltpu
from jax.experimental.pallas import tpu_sc as plsc

_N = 10000
_E = 160000
_D = 256
_HC = 256
_CH = 128
_NS = 16
_CB = 32
_EPS = _E // _NS
_NCH = _EPS // _CB
_ER = _EPS - _NCH * _CB
_NPS = 624
_RB = 8
_ACCW = 144
_BN = 1000
_GI = _N // _BN


def _pre_body(x_ref, g_ref, b_ref, wq_ref, bq_ref, wk_ref, bk_ref,
              wv_ref, bv_ref, ws_ref, bs_ref,
              q_ref, k_ref, v_ref, xr_ref):
    xb = x_ref[...]
    mu = jnp.mean(xb, axis=1, keepdims=True)
    xc = xb - mu
    var = jnp.mean(xc * xc, axis=1, keepdims=True)
    xn = xc * lax.rsqrt(var + 1e-5) * g_ref[...] + b_ref[...]
    q_ref[...] = jnp.dot(xn, wq_ref[...], preferred_element_type=jnp.float32) + bq_ref[...]
    k_ref[...] = jnp.dot(xn, wk_ref[...], preferred_element_type=jnp.float32) + bk_ref[...]
    v_ref[...] = jnp.dot(xn, wv_ref[...], preferred_element_type=jnp.float32) + bv_ref[...]
    xr_ref[...] = jnp.dot(xn, ws_ref[...], preferred_element_type=jnp.float32) + bs_ref[...]


def _pre(x, ln1_g, ln1_b, Wq, bq, Wk, bk, Wv, bv, Wskip, bskip):
    row = pl.BlockSpec((_BN, _D), lambda i, c: (i, 0))
    wcol = pl.BlockSpec((_D, _CH), lambda i, c: (0, c))
    bcol = pl.BlockSpec((1, _CH), lambda i, c: (0, c))
    vec = pl.BlockSpec((1, _D), lambda i, c: (0, 0))
    qkv_out = pl.BlockSpec((_BN, _CH), lambda i, c: (c * _GI + i, 0))
    f32 = jnp.float32
    return pl.pallas_call(
        _pre_body,
        grid=(_GI, 2),
        in_specs=[row, vec, vec, wcol, bcol, wcol, bcol, wcol, bcol, wcol, bcol],
        out_specs=[qkv_out, qkv_out, qkv_out,
                   pl.BlockSpec((_BN, _CH), lambda i, c: (i, c))],
        out_shape=[jax.ShapeDtypeStruct((2 * _N, _CH), f32)] * 3
        + [jax.ShapeDtypeStruct((_N, _D), f32)],
    )(x, ln1_g.reshape(1, _D), ln1_b.reshape(1, _D), Wq, bq.reshape(1, _HC),
      Wk, bk.reshape(1, _HC), Wv, bv.reshape(1, _HC), Wskip, bskip.reshape(1, _HC))


def _relwe_body(rel_ref, we_ref, be_ref, out_ref):
    out_ref[...] = jnp.dot(rel_ref[...], we_ref[...],
                           preferred_element_type=jnp.float32) + be_ref[...]


def _relwe(rel_pad, We, be):
    return pl.pallas_call(
        _relwe_body,
        grid=(2,),
        in_specs=[pl.BlockSpec((24, 64), lambda c: (0, 0)),
                  pl.BlockSpec((64, _CH), lambda c: (0, c)),
                  pl.BlockSpec((1, _CH), lambda c: (0, c))],
        out_specs=pl.BlockSpec((24, _CH), lambda c: (c, 0)),
        out_shape=jax.ShapeDtypeStruct((48, _CH), jnp.float32),
    )(rel_pad, We, be.reshape(1, _HC))


def _edge_body(q2, k2, v2, rw, dst1, src1, et1, out2,
               acc, rwb,
               qbA, kbA, vbA, qbB, kbB, vbB, msgA, msgB,
               bgdA, bgsA, bgeA, bgdB, bgsB, bgeB,
               daA, saA, daB, saB, dsA, dsB,
               dc2, da2, sa2,
               ob, semA, semB, semIA, semIB, semSA, semSB):
    c = lax.axis_index("c")
    s = lax.axis_index("s")
    coff = jnp.full((16,), c * _N, dtype=jnp.int32)
    iota = lax.iota(jnp.int32, 16)
    lo = iota < 8
    inv_sqrt_c = 0.125

    pltpu.sync_copy(rw.at[pl.ds(c * 24, 24)], rwb)

    def zrow(r, _):
        for j in range(_ACCW // 16):
            msgA[r, pl.ds(j * 16, 16)] = jnp.zeros((16,), jnp.float32)
        return 0
    lax.fori_loop(0, _RB, zrow, 0)

    def zblk(b, _):
        pltpu.sync_copy(msgA.at[pl.ds(0, _RB)],
                        acc.at[pl.ds(s * _NPS + b * _RB, _RB)])
        return 0
    lax.fori_loop(0, _NPS // _RB, zblk, 0)

    @pl.when(s == 0)
    def _():
        pltpu.sync_copy(msgA.at[pl.ds(0, _RB)], acc.at[pl.ds(_NS * _NPS, _RB)])
        pltpu.sync_copy(msgA.at[pl.ds(0, _RB)],
                        acc.at[pl.ds(_NS * _NPS + _RB, _RB)])

    plsc.subcore_barrier()

    ebase = s * _EPS

    def load_idx_pair(pair, bgd, bgs, bge, sem):
        b0 = ebase + pair * 2 * _CB
        pltpu.async_copy(dst1.at[pl.ds(b0, 2 * _CB)], bgd, sem)
        pltpu.async_copy(src1.at[pl.ds(b0, 2 * _CB)], bgs, sem)
        pltpu.async_copy(et1.at[pl.ds(b0, 2 * _CB)], bge.at[pl.ds(0, 2 * _CB)], sem)

    def wait_idx(bgd, bgs, bge, sem):
        pltpu.make_async_copy(dst1.at[pl.ds(0, 2 * _CB)], bgd, sem).wait()
        pltpu.make_async_copy(src1.at[pl.ds(0, 2 * _CB)], bgs, sem).wait()
        pltpu.make_async_copy(et1.at[pl.ds(0, 2 * _CB)],
                              bge.at[pl.ds(0, 2 * _CB)], sem).wait()

    def adjust(bgd, bgs, off, da, sa):
        for t in range(_CB // 16):
            sl = pl.ds(t * 16, 16)
            so = pl.ds(off + t * 16, 16)
            da[sl] = bgd[so] + coff
            sa[sl] = bgs[so] + coff

    def issue_gathers(da, sa, qb, kb, vb, sem):
        pltpu.async_copy(q2.at[da], qb, sem)
        pltpu.async_copy(k2.at[sa], kb, sem)
        pltpu.async_copy(v2.at[sa], vb, sem)

    def wait_gathers(da, sa, qb, kb, vb, sem):
        pltpu.make_async_copy(q2.at[da], qb, sem).wait()
        pltpu.make_async_copy(k2.at[sa], kb, sem).wait()
        pltpu.make_async_copy(v2.at[sa], vb, sem).wait()

    def wait_scatter(msg, ds, sem):
        pltpu.make_async_copy(msg, acc.at[ds], sem).wait()

    def compute(nedge, bge, eoff, qb, kb, vb, msg):
        @plsc.parallel_loop(0, nedge, 1, unroll=6)
        def edge(e):
            t = bge[pl.ds(eoff + e, 16)][0]
            a0 = jnp.zeros((16,), jnp.float32)
            a1 = jnp.zeros((16,), jnp.float32)
            vs = []
            for j in range(8):
                sl = pl.ds(j * 16, 16)
                ee = rwb[t, sl]
                kj = kb[e, sl] + ee
                vj = vb[e, sl] + ee
                qj = qb[e, sl]
                if j < 4:
                    a0 = a0 + qj * kj
                else:
                    a1 = a1 + qj * kj
                vs.append(vj)
            ex0 = jnp.exp(jnp.full((16,), jnp.sum(a0) * inv_sqrt_c, jnp.float32))
            ex1 = jnp.exp(jnp.full((16,), jnp.sum(a1) * inv_sqrt_c, jnp.float32))
            for j in range(8):
                msg[e, pl.ds(j * 16, 16)] = vs[j] * (ex0 if j < 4 else ex1)
            msg[e, pl.ds(128, 16)] = jnp.where(lo, ex0, ex1)

    def copy_ds(bgd, off, ds):
        for t in range(_CB // 16):
            ds[pl.ds(t * 16, 16)] = bgd[pl.ds(off + t * 16, 16)]

    def issue_scatter(msg, ds, sem):
        pltpu.async_copy(msg, acc.at[ds], sem, add=True)

    pltpu.sync_copy(dst1.at[pl.ds(ebase, 2 * _CB)], bgdA)
    pltpu.sync_copy(src1.at[pl.ds(ebase, 2 * _CB)], bgsA)
    pltpu.sync_copy(et1.at[pl.ds(ebase, 2 * _CB)], bgeA.at[pl.ds(0, 2 * _CB)])
    load_idx_pair(1, bgdB, bgsB, bgeB, semIB)
    adjust(bgdA, bgsA, 0, daA, saA)
    issue_gathers(daA, saA, qbA, kbA, vbA, semA)

    def quad(i, _):
        adjust(bgdA, bgsA, _CB, daB, saB)
        issue_gathers(daB, saB, qbB, kbB, vbB, semB)
        wait_gathers(daA, saA, qbA, kbA, vbA, semA)

        @pl.when(i > 0)
        def _():
            wait_scatter(msgA, dsA, semSA)
        copy_ds(bgdA, 0, dsA)
        compute(_CB, bgeA, 0, qbA, kbA, vbA, msgA)
        issue_scatter(msgA, dsA, semSA)
        wait_idx(bgdB, bgsB, bgeB, semIB)
        adjust(bgdB, bgsB, 0, daA, saA)
        issue_gathers(daA, saA, qbA, kbA, vbA, semA)
        wait_gathers(daB, saB, qbB, kbB, vbB, semB)

        @pl.when(i > 0)
        def _():
            wait_scatter(msgB, dsB, semSB)
        copy_ds(bgdA, _CB, dsB)
        compute(_CB, bgeA, _CB, qbB, kbB, vbB, msgB)
        issue_scatter(msgB, dsB, semSB)
        load_idx_pair(2 * i + 2, bgdA, bgsA, bgeA, semIA)
        adjust(bgdB, bgsB, _CB, daB, saB)
        issue_gathers(daB, saB, qbB, kbB, vbB, semB)
        wait_gathers(daA, saA, qbA, kbA, vbA, semA)
        wait_scatter(msgA, dsA, semSA)
        copy_ds(bgdB, 0, dsA)
        compute(_CB, bgeB, 0, qbA, kbA, vbA, msgA)
        issue_scatter(msgA, dsA, semSA)
        wait_gathers(daB, saB, qbB, kbB, vbB, semB)
        wait_scatter(msgB, dsB, semSB)
        copy_ds(bgdB, _CB, dsB)
        compute(_CB, bgeB, _CB, qbB, kbB, vbB, msgB)
        issue_scatter(msgB, dsB, semSB)
        load_idx_pair(2 * i + 3, bgdB, bgsB, bgeB, semIB)
        wait_idx(bgdA, bgsA, bgeA, semIA)
        adjust(bgdA, bgsA, 0, daA, saA)
        issue_gathers(daA, saA, qbA, kbA, vbA, semA)
        return 0
    lax.fori_loop(0, (_NCH - 4) // 4, quad, 0)

    adjust(bgdA, bgsA, _CB, daB, saB)
    issue_gathers(daB, saB, qbB, kbB, vbB, semB)
    wait_gathers(daA, saA, qbA, kbA, vbA, semA)
    wait_scatter(msgA, dsA, semSA)
    copy_ds(bgdA, 0, dsA)
    compute(_CB, bgeA, 0, qbA, kbA, vbA, msgA)
    issue_scatter(msgA, dsA, semSA)
    wait_idx(bgdB, bgsB, bgeB, semIB)
    adjust(bgdB, bgsB, 0, daA, saA)
    issue_gathers(daA, saA, qbA, kbA, vbA, semA)
    wait_gathers(daB, saB, qbB, kbB, vbB, semB)
    wait_scatter(msgB, dsB, semSB)
    copy_ds(bgdA, _CB, dsB)
    compute(_CB, bgeA, _CB, qbB, kbB, vbB, msgB)
    issue_scatter(msgB, dsB, semSB)
    adjust(bgdB, bgsB, _CB, daB, saB)
    issue_gathers(daB, saB, qbB, kbB, vbB, semB)
    wait_gathers(daA, saA, qbA, kbA, vbA, semA)
    wait_scatter(msgA, dsA, semSA)
    copy_ds(bgdB, 0, dsA)
    compute(_CB, bgeB, 0, qbA, kbA, vbA, msgA)
    issue_scatter(msgA, dsA, semSA)
    wait_gathers(daB, saB, qbB, kbB, vbB, semB)
    wait_scatter(msgB, dsB, semSB)
    copy_ds(bgdB, _CB, dsB)
    compute(_CB, bgeB, _CB, qbB, kbB, vbB, msgB)
    issue_scatter(msgB, dsB, semSB)

    rbase = ebase + _NCH * _CB
    pltpu.sync_copy(dst1.at[pl.ds(rbase, _ER)], dc2)
    pltpu.sync_copy(src1.at[pl.ds(rbase, _ER)], sa2)
    pltpu.sync_copy(et1.at[pl.ds(rbase, _ER)], bgeA.at[pl.ds(0, _ER)])
    da2[...] = dc2[...] + coff
    sa2[...] = sa2[...] + coff
    r1 = pltpu.async_copy(q2.at[da2], qbA.at[pl.ds(0, _ER)], semA)
    r2 = pltpu.async_copy(k2.at[sa2], kbA.at[pl.ds(0, _ER)], semA)
    r3 = pltpu.async_copy(v2.at[sa2], vbA.at[pl.ds(0, _ER)], semA)
    r1.wait()
    r2.wait()
    r3.wait()
    wait_scatter(msgA, dsA, semSA)
    compute(_ER, bgeA, 0, qbA, kbA, vbA, msgA)
    pltpu.sync_copy(msgA.at[pl.ds(0, _ER)], acc.at[dc2], add=True)
    wait_scatter(msgB, dsB, semSB)

    plsc.subcore_barrier()

    def nblock(rbase):
        pltpu.sync_copy(acc.at[pl.ds(rbase, _RB)], msgA.at[pl.ds(0, _RB)])

        def nrow(r, _):
            inv = 1.0 / (msgA[r, pl.ds(128, 16)] + 1e-16)
            rv = lax.rev(inv, dimensions=(0,))
            inv0 = jnp.where(lo, inv, rv)
            inv1 = jnp.where(lo, rv, inv)
            for j in range(8):
                sl = pl.ds(j * 16, 16)
                ob[r, sl] = msgA[r, sl] * (inv0 if j < 4 else inv1)
            return 0
        lax.fori_loop(0, _RB, nrow, 0)
        pltpu.sync_copy(ob, out2.at[pl.ds(c * _N + rbase, _RB)])

    def nblk(b, _):
        nblock(s * _NPS + b * _RB)
        return 0
    lax.fori_loop(0, _NPS // _RB, nblk, 0)

    @pl.when(s == 0)
    def _():
        nblock(_NS * _NPS)
        nblock(_NS * _NPS + _RB)


def _edge(q2, k2, v2, rw, dst1, src1, et1):
    f32, i32 = jnp.float32, jnp.int32
    mesh = plsc.VectorSubcoreMesh(core_axis_name="c", subcore_axis_name="s",
                                  num_cores=2, num_subcores=_NS)
    return pl.kernel(
        _edge_body,
        out_type=jax.ShapeDtypeStruct((2 * _N, _CH), f32),
        mesh=mesh,
        compiler_params=pltpu.CompilerParams(use_tc_tiling_on_sc=False,
                                             needs_layout_passes=False),
        scratch_types=[
            pltpu.VMEM_SHARED((_N, _ACCW), f32),
            pltpu.VMEM((24, _CH), f32),
            pltpu.VMEM((_CB, _CH), f32),
            pltpu.VMEM((_CB, _CH), f32),
            pltpu.VMEM((_CB, _CH), f32),
            pltpu.VMEM((_CB, _CH), f32),
            pltpu.VMEM((_CB, _CH), f32),
            pltpu.VMEM((_CB, _CH), f32),
            pltpu.VMEM((_CB, _ACCW), f32),
            pltpu.VMEM((_CB, _ACCW), f32),
            pltpu.VMEM((2 * _CB,), i32),
            pltpu.VMEM((2 * _CB,), i32),
            pltpu.VMEM((2 * _CB + 16,), i32),
            pltpu.VMEM((2 * _CB,), i32),
            pltpu.VMEM((2 * _CB,), i32),
            pltpu.VMEM((2 * _CB + 16,), i32),
            pltpu.VMEM((_CB,), i32),
            pltpu.VMEM((_CB,), i32),
            pltpu.VMEM((_CB,), i32),
            pltpu.VMEM((_CB,), i32),
            pltpu.VMEM((_CB,), i32),
            pltpu.VMEM((_CB,), i32),
            pltpu.VMEM((_ER,), i32),
            pltpu.VMEM((_ER,), i32),
            pltpu.VMEM((_ER,), i32),
            pltpu.VMEM((_RB, _CH), f32),
            pltpu.SemaphoreType.DMA,
            pltpu.SemaphoreType.DMA,
            pltpu.SemaphoreType.DMA,
            pltpu.SemaphoreType.DMA,
            pltpu.SemaphoreType.DMA,
            pltpu.SemaphoreType.DMA,
        ],
    )(q2, k2, v2, rw, dst1, src1, et1)


def _post_body(olo_ref, ohi_ref, xr_ref, x_ref, wb_ref, g2_ref, b2_ref,
               wf1_ref, bf1_ref, wf2_ref, bf2_ref, o_ref):
    out = jnp.concatenate([olo_ref[...], ohi_ref[...]], axis=1)
    xr = xr_ref[...]
    wb = wb_ref[...]
    gp = (jnp.sum(out * (wb[0:1, :] + wb[2:3, :]), axis=1, keepdims=True)
          + jnp.sum(xr * (wb[1:2, :] - wb[2:3, :]), axis=1, keepdims=True))
    g = jax.nn.sigmoid(gp)
    h = x_ref[...] + g * xr + (1.0 - g) * out
    mu = jnp.mean(h, axis=1, keepdims=True)
    hc = h - mu
    var = jnp.mean(hc * hc, axis=1, keepdims=True)
    hn = hc * lax.rsqrt(var + 1e-5) * g2_ref[...] + b2_ref[...]
    t = jnp.dot(hn, wf1_ref[...], preferred_element_type=jnp.float32) + bf1_ref[...]
    t = 0.5 * t * (1.0 + lax.erf(t * 0.7071067811865476))
    hf = jnp.dot(t, wf2_ref[...], preferred_element_type=jnp.float32) + bf2_ref[...]
    o_ref[...] = h + hf


def _post(olo, ohi, xr, x, Wb3, ln2_g, ln2_b, Wf1, bf1, Wf2, bf2):
    row_h = pl.BlockSpec((_BN, _CH), lambda i: (i, 0))
    row = pl.BlockSpec((_BN, _D), lambda i: (i, 0))

    def full(a, b):
        return pl.BlockSpec((a, b), lambda i: (0, 0))

    return pl.pallas_call(
        _post_body,
        grid=(_GI,),
        in_specs=[row_h, row_h, row, row, full(3, _D), full(1, _D), full(1, _D),
                  full(_D, 4 * _D), full(1, 4 * _D), full(4 * _D, _D), full(1, _D)],
        out_specs=row,
        out_shape=jax.ShapeDtypeStruct((_N, _D), jnp.float32),
    )(olo, ohi, xr, x, Wb3, ln2_g.reshape(1, _D), ln2_b.reshape(1, _D),
      Wf1, bf1.reshape(1, 4 * _D), Wf2, bf2.reshape(1, _D))


def kernel(x, edge_index, edge_type, rel_emb, Wq, bq, Wk, bk, Wv, bv, We, be,
           Wskip, bskip, Wbeta, ln1_g, ln1_b, ln2_g, ln2_b, Wf1, bf1, Wf2, bf2):
    q2, k2, v2, xr = _pre(x, ln1_g, ln1_b, Wq, bq, Wk, bk, Wv, bv, Wskip, bskip)
    rel_pad = jnp.zeros((24, 64), jnp.float32).at[:17].set(rel_emb)
    rw = _relwe(rel_pad, We, be)
    out2 = _edge(q2, k2, v2, rw, edge_index[1], edge_index[0], edge_type)
    return _post(out2[:_N], out2[_N:], xr, x, Wbeta.reshape(3, _D),
                 ln2_g, ln2_b, Wf1, bf1, Wf2, bf2)


# speedup vs baseline: 1.0993x; 1.0993x over previous
import jax
import jax.numpy as jnp
from jax import lax
from jax.experimental import pallas as pl
from jax.experimental.pallas import tpu as pltpu
from jax.experimental.pallas import tpu_sc as plsc

_N = 10000
_E = 160000
_D = 256
_HC = 256
_CH = 128
_NS = 16
_CB = 32
_EPS = _E // _NS
_NCH = _EPS // _CB
_ER = _EPS - _NCH * _CB
_NPS = 624
_RB = 8
_ACCW = 144
_BN = 1000
_GI = _N // _BN


def _pre_body(x_ref, g_ref, b_ref, wq_ref, bq_ref, wk_ref, bk_ref,
              wv_ref, bv_ref, ws_ref, bs_ref,
              q_ref, kv_ref, xr_ref):
    xb = x_ref[...]
    mu = jnp.mean(xb, axis=1, keepdims=True)
    xc = xb - mu
    var = jnp.mean(xc * xc, axis=1, keepdims=True)
    xn = xc * lax.rsqrt(var + 1e-5) * g_ref[...] + b_ref[...]
    q_ref[...] = jnp.dot(xn, wq_ref[...], preferred_element_type=jnp.float32) + bq_ref[...]
    k = jnp.dot(xn, wk_ref[...], preferred_element_type=jnp.float32) + bk_ref[...]
    v = jnp.dot(xn, wv_ref[...], preferred_element_type=jnp.float32) + bv_ref[...]
    kv_ref[...] = jnp.concatenate([k, v], axis=1)
    xr_ref[...] = jnp.dot(xn, ws_ref[...], preferred_element_type=jnp.float32) + bs_ref[...]


def _pre(x, ln1_g, ln1_b, Wq, bq, Wk, bk, Wv, bv, Wskip, bskip):
    row = pl.BlockSpec((_BN, _D), lambda i, c: (i, 0))
    wcol = pl.BlockSpec((_D, _CH), lambda i, c: (0, c))
    bcol = pl.BlockSpec((1, _CH), lambda i, c: (0, c))
    vec = pl.BlockSpec((1, _D), lambda i, c: (0, 0))
    q_out = pl.BlockSpec((_BN, _CH), lambda i, c: (c * _GI + i, 0))
    kv_out = pl.BlockSpec((_BN, 2 * _CH), lambda i, c: (c * _GI + i, 0))
    f32 = jnp.float32
    return pl.pallas_call(
        _pre_body,
        grid=(_GI, 2),
        in_specs=[row, vec, vec, wcol, bcol, wcol, bcol, wcol, bcol, wcol, bcol],
        out_specs=[q_out, kv_out,
                   pl.BlockSpec((_BN, _CH), lambda i, c: (i, c))],
        out_shape=[jax.ShapeDtypeStruct((2 * _N, _CH), f32),
                   jax.ShapeDtypeStruct((2 * _N, 2 * _CH), f32),
                   jax.ShapeDtypeStruct((_N, _D), f32)],
    )(x, ln1_g.reshape(1, _D), ln1_b.reshape(1, _D), Wq, bq.reshape(1, _HC),
      Wk, bk.reshape(1, _HC), Wv, bv.reshape(1, _HC), Wskip, bskip.reshape(1, _HC))


def _relwe_body(rel_ref, we_ref, be_ref, out_ref):
    out_ref[...] = jnp.dot(rel_ref[...], we_ref[...],
                           preferred_element_type=jnp.float32) + be_ref[...]


def _relwe(rel_pad, We, be):
    return pl.pallas_call(
        _relwe_body,
        grid=(2,),
        in_specs=[pl.BlockSpec((24, 64), lambda c: (0, 0)),
                  pl.BlockSpec((64, _CH), lambda c: (0, c)),
                  pl.BlockSpec((1, _CH), lambda c: (0, c))],
        out_specs=pl.BlockSpec((24, _CH), lambda c: (c, 0)),
        out_shape=jax.ShapeDtypeStruct((48, _CH), jnp.float32),
    )(rel_pad, We, be.reshape(1, _HC))


def _edge_body(q2, kv2, rw, dst1, src1, et1, out2,
               acc, rwb,
               qbA, kvbA, qbB, kvbB, msgA, msgB,
               bgdA, bgsA, bgeA, bgdB, bgsB, bgeB,
               daA, saA, daB, saB, dsA, dsB,
               dc2, da2, sa2,
               ob, semA, semB, semIA, semIB, semSA, semSB):
    c = lax.axis_index("c")
    s = lax.axis_index("s")
    coff = jnp.full((16,), c * _N, dtype=jnp.int32)
    iota = lax.iota(jnp.int32, 16)
    lo = iota < 8
    inv_sqrt_c = 0.125

    pltpu.sync_copy(rw.at[pl.ds(c * 24, 24)], rwb)

    def zrow(r, _):
        for j in range(_ACCW // 16):
            msgA[r, pl.ds(j * 16, 16)] = jnp.zeros((16,), jnp.float32)
        return 0
    lax.fori_loop(0, _RB, zrow, 0)

    def zblk(b, _):
        pltpu.sync_copy(msgA.at[pl.ds(0, _RB)],
                        acc.at[pl.ds(s * _NPS + b * _RB, _RB)])
        return 0
    lax.fori_loop(0, _NPS // _RB, zblk, 0)

    @pl.when(s == 0)
    def _():
        pltpu.sync_copy(msgA.at[pl.ds(0, _RB)], acc.at[pl.ds(_NS * _NPS, _RB)])
        pltpu.sync_copy(msgA.at[pl.ds(0, _RB)],
                        acc.at[pl.ds(_NS * _NPS + _RB, _RB)])

    plsc.subcore_barrier()

    ebase = s * _EPS

    def load_idx_pair(pair, bgd, bgs, bge, sem):
        b0 = ebase + pair * 2 * _CB
        pltpu.async_copy(dst1.at[pl.ds(b0, 2 * _CB)], bgd, sem)
        pltpu.async_copy(src1.at[pl.ds(b0, 2 * _CB)], bgs, sem)
        pltpu.async_copy(et1.at[pl.ds(b0, 2 * _CB)], bge.at[pl.ds(0, 2 * _CB)], sem)

    def wait_idx(bgd, bgs, bge, sem):
        pltpu.make_async_copy(dst1.at[pl.ds(0, 2 * _CB)], bgd, sem).wait()
        pltpu.make_async_copy(src1.at[pl.ds(0, 2 * _CB)], bgs, sem).wait()
        pltpu.make_async_copy(et1.at[pl.ds(0, 2 * _CB)],
                              bge.at[pl.ds(0, 2 * _CB)], sem).wait()

    def adjust(bgd, bgs, off, da, sa):
        for t in range(_CB // 16):
            sl = pl.ds(t * 16, 16)
            so = pl.ds(off + t * 16, 16)
            da[sl] = bgd[so] + coff
            sa[sl] = bgs[so] + coff

    def issue_gathers(da, sa, qb, kvb, sem):
        pltpu.async_copy(q2.at[da], qb, sem)
        pltpu.async_copy(kv2.at[sa], kvb, sem)

    def wait_gathers(da, sa, qb, kvb, sem):
        pltpu.make_async_copy(q2.at[da], qb, sem).wait()
        pltpu.make_async_copy(kv2.at[sa], kvb, sem).wait()

    def wait_scatter(msg, ds, sem):
        pltpu.make_async_copy(msg, acc.at[ds], sem).wait()

    def compute(nedge, bge, eoff, qb, kvb, msg):
        @plsc.parallel_loop(0, nedge, 1, unroll=4)
        def edge(e):
            t = bge[pl.ds(eoff + e, 16)][0]
            a0 = jnp.zeros((16,), jnp.float32)
            a1 = jnp.zeros((16,), jnp.float32)
            vs = []
            for j in range(8):
                sl = pl.ds(j * 16, 16)
                ee = rwb[t, sl]
                kj = kvb[e, sl] + ee
                vj = kvb[e, pl.ds(128 + j * 16, 16)] + ee
                qj = qb[e, sl]
                if j < 4:
                    a0 = a0 + qj * kj
                else:
                    a1 = a1 + qj * kj
                vs.append(vj)
            ex0 = jnp.exp(jnp.full((16,), jnp.sum(a0) * inv_sqrt_c, jnp.float32))
            ex1 = jnp.exp(jnp.full((16,), jnp.sum(a1) * inv_sqrt_c, jnp.float32))
            for j in range(8):
                msg[e, pl.ds(j * 16, 16)] = vs[j] * (ex0 if j < 4 else ex1)
            msg[e, pl.ds(128, 16)] = jnp.where(lo, ex0, ex1)

    def copy_ds(bgd, off, ds):
        for t in range(_CB // 16):
            ds[pl.ds(t * 16, 16)] = bgd[pl.ds(off + t * 16, 16)]

    def issue_scatter(msg, ds, sem):
        pltpu.async_copy(msg, acc.at[ds], sem, add=True)

    pltpu.sync_copy(dst1.at[pl.ds(ebase, 2 * _CB)], bgdA)
    pltpu.sync_copy(src1.at[pl.ds(ebase, 2 * _CB)], bgsA)
    pltpu.sync_copy(et1.at[pl.ds(ebase, 2 * _CB)], bgeA.at[pl.ds(0, 2 * _CB)])
    load_idx_pair(1, bgdB, bgsB, bgeB, semIB)
    adjust(bgdA, bgsA, 0, daA, saA)
    issue_gathers(daA, saA, qbA, kvbA, semA)

    def quad(i, _):
        adjust(bgdA, bgsA, _CB, daB, saB)
        issue_gathers(daB, saB, qbB, kvbB, semB)
        wait_gathers(daA, saA, qbA, kvbA, semA)

        @pl.when(i > 0)
        def _():
            wait_scatter(msgA, dsA, semSA)
        copy_ds(bgdA, 0, dsA)
        compute(_CB, bgeA, 0, qbA, kvbA, msgA)
        issue_scatter(msgA, dsA, semSA)
        wait_idx(bgdB, bgsB, bgeB, semIB)
        adjust(bgdB, bgsB, 0, daA, saA)
        issue_gathers(daA, saA, qbA, kvbA, semA)
        wait_gathers(daB, saB, qbB, kvbB, semB)

        @pl.when(i > 0)
        def _():
            wait_scatter(msgB, dsB, semSB)
        copy_ds(bgdA, _CB, dsB)
        compute(_CB, bgeA, _CB, qbB, kvbB, msgB)
        issue_scatter(msgB, dsB, semSB)
        load_idx_pair(2 * i + 2, bgdA, bgsA, bgeA, semIA)
        adjust(bgdB, bgsB, _CB, daB, saB)
        issue_gathers(daB, saB, qbB, kvbB, semB)
        wait_gathers(daA, saA, qbA, kvbA, semA)
        wait_scatter(msgA, dsA, semSA)
        copy_ds(bgdB, 0, dsA)
        compute(_CB, bgeB, 0, qbA, kvbA, msgA)
        issue_scatter(msgA, dsA, semSA)
        wait_gathers(daB, saB, qbB, kvbB, semB)
        wait_scatter(msgB, dsB, semSB)
        copy_ds(bgdB, _CB, dsB)
        compute(_CB, bgeB, _CB, qbB, kvbB, msgB)
        issue_scatter(msgB, dsB, semSB)
        load_idx_pair(2 * i + 3, bgdB, bgsB, bgeB, semIB)
        wait_idx(bgdA, bgsA, bgeA, semIA)
        adjust(bgdA, bgsA, 0, daA, saA)
        issue_gathers(daA, saA, qbA, kvbA, semA)
        return 0
    lax.fori_loop(0, (_NCH - 4) // 4, quad, 0)

    adjust(bgdA, bgsA, _CB, daB, saB)
    issue_gathers(daB, saB, qbB, kvbB, semB)
    wait_gathers(daA, saA, qbA, kvbA, semA)
    wait_scatter(msgA, dsA, semSA)
    copy_ds(bgdA, 0, dsA)
    compute(_CB, bgeA, 0, qbA, kvbA, msgA)
    issue_scatter(msgA, dsA, semSA)
    wait_idx(bgdB, bgsB, bgeB, semIB)
    adjust(bgdB, bgsB, 0, daA, saA)
    issue_gathers(daA, saA, qbA, kvbA, semA)
    wait_gathers(daB, saB, qbB, kvbB, semB)
    wait_scatter(msgB, dsB, semSB)
    copy_ds(bgdA, _CB, dsB)
    compute(_CB, bgeA, _CB, qbB, kvbB, msgB)
    issue_scatter(msgB, dsB, semSB)
    adjust(bgdB, bgsB, _CB, daB, saB)
    issue_gathers(daB, saB, qbB, kvbB, semB)
    wait_gathers(daA, saA, qbA, kvbA, semA)
    wait_scatter(msgA, dsA, semSA)
    copy_ds(bgdB, 0, dsA)
    compute(_CB, bgeB, 0, qbA, kvbA, msgA)
    issue_scatter(msgA, dsA, semSA)
    wait_gathers(daB, saB, qbB, kvbB, semB)
    wait_scatter(msgB, dsB, semSB)
    copy_ds(bgdB, _CB, dsB)
    compute(_CB, bgeB, _CB, qbB, kvbB, msgB)
    issue_scatter(msgB, dsB, semSB)

    rbase = ebase + _NCH * _CB
    pltpu.sync_copy(dst1.at[pl.ds(rbase, _ER)], dc2)
    pltpu.sync_copy(src1.at[pl.ds(rbase, _ER)], sa2)
    pltpu.sync_copy(et1.at[pl.ds(rbase, _ER)], bgeA.at[pl.ds(0, _ER)])
    da2[...] = dc2[...] + coff
    sa2[...] = sa2[...] + coff
    r1 = pltpu.async_copy(q2.at[da2], qbA.at[pl.ds(0, _ER)], semA)
    r2 = pltpu.async_copy(kv2.at[sa2], kvbA.at[pl.ds(0, _ER)], semA)
    r1.wait()
    r2.wait()
    wait_scatter(msgA, dsA, semSA)
    compute(_ER, bgeA, 0, qbA, kvbA, msgA)
    pltpu.sync_copy(msgA.at[pl.ds(0, _ER)], acc.at[dc2], add=True)
    wait_scatter(msgB, dsB, semSB)

    plsc.subcore_barrier()

    def nblock(rbase):
        pltpu.sync_copy(acc.at[pl.ds(rbase, _RB)], msgA.at[pl.ds(0, _RB)])

        def nrow(r, _):
            inv = 1.0 / (msgA[r, pl.ds(128, 16)] + 1e-16)
            rv = lax.rev(inv, dimensions=(0,))
            inv0 = jnp.where(lo, inv, rv)
            inv1 = jnp.where(lo, rv, inv)
            for j in range(8):
                sl = pl.ds(j * 16, 16)
                ob[r, sl] = msgA[r, sl] * (inv0 if j < 4 else inv1)
            return 0
        lax.fori_loop(0, _RB, nrow, 0)
        pltpu.sync_copy(ob, out2.at[pl.ds(c * _N + rbase, _RB)])

    def nblk(b, _):
        nblock(s * _NPS + b * _RB)
        return 0
    lax.fori_loop(0, _NPS // _RB, nblk, 0)

    @pl.when(s == 0)
    def _():
        nblock(_NS * _NPS)
        nblock(_NS * _NPS + _RB)


def _edge(q2, kv2, rw, dst1, src1, et1):
    f32, i32 = jnp.float32, jnp.int32
    mesh = plsc.VectorSubcoreMesh(core_axis_name="c", subcore_axis_name="s",
                                  num_cores=2, num_subcores=_NS)
    return pl.kernel(
        _edge_body,
        out_type=jax.ShapeDtypeStruct((2 * _N, _CH), f32),
        mesh=mesh,
        compiler_params=pltpu.CompilerParams(use_tc_tiling_on_sc=False,
                                             needs_layout_passes=False),
        scratch_types=[
            pltpu.VMEM_SHARED((_N, _ACCW), f32),
            pltpu.VMEM((24, _CH), f32),
            pltpu.VMEM((_CB, _CH), f32),
            pltpu.VMEM((_CB, 2 * _CH), f32),
            pltpu.VMEM((_CB, _CH), f32),
            pltpu.VMEM((_CB, 2 * _CH), f32),
            pltpu.VMEM((_CB, _ACCW), f32),
            pltpu.VMEM((_CB, _ACCW), f32),
            pltpu.VMEM((2 * _CB,), i32),
            pltpu.VMEM((2 * _CB,), i32),
            pltpu.VMEM((2 * _CB + 16,), i32),
            pltpu.VMEM((2 * _CB,), i32),
            pltpu.VMEM((2 * _CB,), i32),
            pltpu.VMEM((2 * _CB + 16,), i32),
            pltpu.VMEM((_CB,), i32),
            pltpu.VMEM((_CB,), i32),
            pltpu.VMEM((_CB,), i32),
            pltpu.VMEM((_CB,), i32),
            pltpu.VMEM((_CB,), i32),
            pltpu.VMEM((_CB,), i32),
            pltpu.VMEM((_ER,), i32),
            pltpu.VMEM((_ER,), i32),
            pltpu.VMEM((_ER,), i32),
            pltpu.VMEM((_RB, _CH), f32),
            pltpu.SemaphoreType.DMA,
            pltpu.SemaphoreType.DMA,
            pltpu.SemaphoreType.DMA,
            pltpu.SemaphoreType.DMA,
            pltpu.SemaphoreType.DMA,
            pltpu.SemaphoreType.DMA,
        ],
    )(q2, kv2, rw, dst1, src1, et1)


def _post_body(olo_ref, ohi_ref, xr_ref, x_ref, wb_ref, g2_ref, b2_ref,
               wf1_ref, bf1_ref, wf2_ref, bf2_ref, o_ref):
    out = jnp.concatenate([olo_ref[...], ohi_ref[...]], axis=1)
    xr = xr_ref[...]
    wb = wb_ref[...]
    gp = (jnp.sum(out * (wb[0:1, :] + wb[2:3, :]), axis=1, keepdims=True)
          + jnp.sum(xr * (wb[1:2, :] - wb[2:3, :]), axis=1, keepdims=True))
    g = jax.nn.sigmoid(gp)
    h = x_ref[...] + g * xr + (1.0 - g) * out
    mu = jnp.mean(h, axis=1, keepdims=True)
    hc = h - mu
    var = jnp.mean(hc * hc, axis=1, keepdims=True)
    hn = hc * lax.rsqrt(var + 1e-5) * g2_ref[...] + b2_ref[...]
    t = jnp.dot(hn, wf1_ref[...], preferred_element_type=jnp.float32) + bf1_ref[...]
    t = 0.5 * t * (1.0 + lax.erf(t * 0.7071067811865476))
    hf = jnp.dot(t, wf2_ref[...], preferred_element_type=jnp.float32) + bf2_ref[...]
    o_ref[...] = h + hf


def _post(olo, ohi, xr, x, Wb3, ln2_g, ln2_b, Wf1, bf1, Wf2, bf2):
    row_h = pl.BlockSpec((_BN, _CH), lambda i: (i, 0))
    row = pl.BlockSpec((_BN, _D), lambda i: (i, 0))

    def full(a, b):
        return pl.BlockSpec((a, b), lambda i: (0, 0))

    return pl.pallas_call(
        _post_body,
        grid=(_GI,),
        in_specs=[row_h, row_h, row, row, full(3, _D), full(1, _D), full(1, _D),
                  full(_D, 4 * _D), full(1, 4 * _D), full(4 * _D, _D), full(1, _D)],
        out_specs=row,
        out_shape=jax.ShapeDtypeStruct((_N, _D), jnp.float32),
    )(olo, ohi, xr, x, Wb3, ln2_g.reshape(1, _D), ln2_b.reshape(1, _D),
      Wf1, bf1.reshape(1, 4 * _D), Wf2, bf2.reshape(1, _D))


def kernel(x, edge_index, edge_type, rel_emb, Wq, bq, Wk, bk, Wv, bv, We, be,
           Wskip, bskip, Wbeta, ln1_g, ln1_b, ln2_g, ln2_b, Wf1, bf1, Wf2, bf2):
    q2, kv2, xr = _pre(x, ln1_g, ln1_b, Wq, bq, Wk, bk, Wv, bv, Wskip, bskip)
    rel_pad = jnp.zeros((24, 64), jnp.float32).at[:17].set(rel_emb)
    rw = _relwe(rel_pad, We, be)
    out2 = _edge(q2, kv2, rw, edge_index[1], edge_index[0], edge_type)
    return _post(out2[:_N], out2[_N:], xr, x, Wbeta.reshape(3, _D),
                 ln2_g, ln2_b, Wf1, bf1, Wf2, bf2)


# speedup vs baseline: 1.1196x; 1.0185x over previous
import jax
import jax.numpy as jnp
from jax import lax
from jax.experimental import pallas as pl
from jax.experimental.pallas import tpu as pltpu
from jax.experimental.pallas import tpu_sc as plsc

_N = 10000
_E = 160000
_D = 256
_HC = 256
_CH = 128
_NS = 16
_CB = 32
_EPS = _E // _NS
_NCH = _EPS // _CB
_ER = _EPS - _NCH * _CB
_NPS = 624
_RB = 8
_ACCW = 144
_BN = 1000
_GI = _N // _BN


def _pre_body(x_ref, g_ref, b_ref, wq_ref, bq_ref, wk_ref, bk_ref,
              wv_ref, bv_ref, ws_ref, bs_ref,
              q_ref, k_ref, v_ref, xr_ref):
    xb = x_ref[...]
    mu = jnp.mean(xb, axis=1, keepdims=True)
    xc = xb - mu
    var = jnp.mean(xc * xc, axis=1, keepdims=True)
    xn = xc * lax.rsqrt(var + 1e-5) * g_ref[...] + b_ref[...]
    q_ref[...] = jnp.dot(xn, wq_ref[...], preferred_element_type=jnp.float32) + bq_ref[...]
    k_ref[...] = jnp.dot(xn, wk_ref[...], preferred_element_type=jnp.float32) + bk_ref[...]
    v_ref[...] = jnp.dot(xn, wv_ref[...], preferred_element_type=jnp.float32) + bv_ref[...]
    xr_ref[...] = jnp.dot(xn, ws_ref[...], preferred_element_type=jnp.float32) + bs_ref[...]


def _pre(x, ln1_g, ln1_b, Wq, bq, Wk, bk, Wv, bv, Wskip, bskip):
    row = pl.BlockSpec((_BN, _D), lambda i, c: (i, 0))
    wcol = pl.BlockSpec((_D, _CH), lambda i, c: (0, c))
    bcol = pl.BlockSpec((1, _CH), lambda i, c: (0, c))
    vec = pl.BlockSpec((1, _D), lambda i, c: (0, 0))
    q_out = pl.BlockSpec((_BN, _CH), lambda i, c: (c * _GI + i, 0))
    f32 = jnp.float32
    return pl.pallas_call(
        _pre_body,
        grid=(_GI, 2),
        in_specs=[row, vec, vec, wcol, bcol, wcol, bcol, wcol, bcol, wcol, bcol],
        out_specs=[q_out, q_out, q_out,
                   pl.BlockSpec((_BN, _CH), lambda i, c: (i, c))],
        out_shape=[jax.ShapeDtypeStruct((2 * _N, _CH), f32)] * 3
        + [jax.ShapeDtypeStruct((_N, _D), f32)],
    )(x, ln1_g.reshape(1, _D), ln1_b.reshape(1, _D), Wq, bq.reshape(1, _HC),
      Wk, bk.reshape(1, _HC), Wv, bv.reshape(1, _HC), Wskip, bskip.reshape(1, _HC))


def _relwe_body(rel_ref, we_ref, be_ref, out_ref):
    out_ref[...] = jnp.dot(rel_ref[...], we_ref[...],
                           preferred_element_type=jnp.float32) + be_ref[...]


def _relwe(rel_pad, We, be):
    return pl.pallas_call(
        _relwe_body,
        grid=(2,),
        in_specs=[pl.BlockSpec((24, 64), lambda c: (0, 0)),
                  pl.BlockSpec((64, _CH), lambda c: (0, c)),
                  pl.BlockSpec((1, _CH), lambda c: (0, c))],
        out_specs=pl.BlockSpec((24, _CH), lambda c: (c, 0)),
        out_shape=jax.ShapeDtypeStruct((48, _CH), jnp.float32),
    )(rel_pad, We, be.reshape(1, _HC))


def _edge_body(q2, k2, v2, rw, dst1, src1, et1, out2,
               acc, rwb,
               qbA, kbA, vbA, qbB, kbB, vbB, msgA, msgB,
               bgdA, bgsA, bgeA, bgdB, bgsB, bgeB,
               daA, saA, daB, saB, dsA, dsB,
               dc2, da2, sa2,
               ob, semA, semB, semIA, semIB, semSA, semSB):
    c = lax.axis_index("c")
    s = lax.axis_index("s")
    coff = jnp.full((16,), c * _N, dtype=jnp.int32)
    iota = lax.iota(jnp.int32, 16)
    lo = iota < 8
    inv_sqrt_c = 0.125

    pltpu.sync_copy(rw.at[pl.ds(c * 24, 24)], rwb)

    def zrow(r, _):
        for j in range(_ACCW // 16):
            msgA[r, pl.ds(j * 16, 16)] = jnp.zeros((16,), jnp.float32)
        return 0
    lax.fori_loop(0, _RB, zrow, 0)

    def zblk(b, _):
        pltpu.sync_copy(msgA.at[pl.ds(0, _RB)],
                        acc.at[pl.ds(s * _NPS + b * _RB, _RB)])
        return 0
    lax.fori_loop(0, _NPS // _RB, zblk, 0)

    @pl.when(s == 0)
    def _():
        pltpu.sync_copy(msgA.at[pl.ds(0, _RB)], acc.at[pl.ds(_NS * _NPS, _RB)])
        pltpu.sync_copy(msgA.at[pl.ds(0, _RB)],
                        acc.at[pl.ds(_NS * _NPS + _RB, _RB)])

    plsc.subcore_barrier()

    ebase = s * _EPS

    def load_idx_pair(pair, bgd, bgs, bge, sem):
        b0 = ebase + pair * 2 * _CB
        pltpu.async_copy(dst1.at[pl.ds(b0, 2 * _CB)], bgd, sem)
        pltpu.async_copy(src1.at[pl.ds(b0, 2 * _CB)], bgs, sem)
        pltpu.async_copy(et1.at[pl.ds(b0, 2 * _CB)], bge.at[pl.ds(0, 2 * _CB)], sem)

    def wait_idx(bgd, bgs, bge, sem):
        pltpu.make_async_copy(dst1.at[pl.ds(0, 2 * _CB)], bgd, sem).wait()
        pltpu.make_async_copy(src1.at[pl.ds(0, 2 * _CB)], bgs, sem).wait()
        pltpu.make_async_copy(et1.at[pl.ds(0, 2 * _CB)],
                              bge.at[pl.ds(0, 2 * _CB)], sem).wait()

    def adjust(bgd, bgs, off, da, sa):
        for t in range(_CB // 16):
            sl = pl.ds(t * 16, 16)
            so = pl.ds(off + t * 16, 16)
            da[sl] = bgd[so] + coff
            sa[sl] = bgs[so] + coff

    def issue_gathers(da, sa, qb, kb, vb, sem):
        pltpu.async_copy(q2.at[da], qb, sem)
        pltpu.async_copy(k2.at[sa], kb, sem)
        pltpu.async_copy(v2.at[sa], vb, sem)

    def wait_gathers(da, sa, qb, kb, vb, sem):
        pltpu.make_async_copy(q2.at[da], qb, sem).wait()
        pltpu.make_async_copy(k2.at[sa], kb, sem).wait()
        pltpu.make_async_copy(v2.at[sa], vb, sem).wait()

    def wait_scatter(msg, ds, sem):
        pltpu.make_async_copy(msg, acc.at[ds], sem).wait()

    def compute(nedge, bge, eoff, qb, kb, vb, msg):
        @plsc.parallel_loop(0, nedge, 1, unroll=4)
        def edge(e):
            t = bge[pl.ds(eoff + e, 16)][0]
            a0 = jnp.zeros((16,), jnp.float32)
            a1 = jnp.zeros((16,), jnp.float32)
            vs = []
            for j in range(8):
                sl = pl.ds(j * 16, 16)
                ee = rwb[t, sl]
                kj = kb[e, sl] + ee
                vj = vb[e, sl] + ee
                qj = qb[e, sl]
                if j < 4:
                    a0 = a0 + qj * kj
                else:
                    a1 = a1 + qj * kj
                vs.append(vj)
            ex0 = jnp.exp(jnp.full((16,), jnp.sum(a0) * inv_sqrt_c, jnp.float32))
            ex1 = jnp.exp(jnp.full((16,), jnp.sum(a1) * inv_sqrt_c, jnp.float32))
            for j in range(8):
                msg[e, pl.ds(j * 16, 16)] = vs[j] * (ex0 if j < 4 else ex1)
            msg[e, pl.ds(128, 16)] = jnp.where(lo, ex0, ex1)

    def copy_ds(bgd, off, ds):
        for t in range(_CB // 16):
            ds[pl.ds(t * 16, 16)] = bgd[pl.ds(off + t * 16, 16)]

    def issue_scatter(msg, ds, sem):
        pltpu.async_copy(msg, acc.at[ds], sem, add=True)

    pltpu.sync_copy(dst1.at[pl.ds(ebase, 2 * _CB)], bgdA)
    pltpu.sync_copy(src1.at[pl.ds(ebase, 2 * _CB)], bgsA)
    pltpu.sync_copy(et1.at[pl.ds(ebase, 2 * _CB)], bgeA.at[pl.ds(0, 2 * _CB)])
    load_idx_pair(1, bgdB, bgsB, bgeB, semIB)
    adjust(bgdA, bgsA, 0, daA, saA)
    issue_gathers(daA, saA, qbA, kbA, vbA, semA)

    def quad(i, _):
        adjust(bgdA, bgsA, _CB, daB, saB)
        issue_gathers(daB, saB, qbB, kbB, vbB, semB)
        wait_gathers(daA, saA, qbA, kbA, vbA, semA)

        @pl.when(i > 0)
        def _():
            wait_scatter(msgA, dsA, semSA)
        copy_ds(bgdA, 0, dsA)
        compute(_CB, bgeA, 0, qbA, kbA, vbA, msgA)
        issue_scatter(msgA, dsA, semSA)
        wait_idx(bgdB, bgsB, bgeB, semIB)
        adjust(bgdB, bgsB, 0, daA, saA)
        issue_gathers(daA, saA, qbA, kbA, vbA, semA)
        wait_gathers(daB, saB, qbB, kbB, vbB, semB)

        @pl.when(i > 0)
        def _():
            wait_scatter(msgB, dsB, semSB)
        copy_ds(bgdA, _CB, dsB)
        compute(_CB, bgeA, _CB, qbB, kbB, vbB, msgB)
        issue_scatter(msgB, dsB, semSB)
        load_idx_pair(2 * i + 2, bgdA, bgsA, bgeA, semIA)
        adjust(bgdB, bgsB, _CB, daB, saB)
        issue_gathers(daB, saB, qbB, kbB, vbB, semB)
        wait_gathers(daA, saA, qbA, kbA, vbA, semA)
        wait_scatter(msgA, dsA, semSA)
        copy_ds(bgdB, 0, dsA)
        compute(_CB, bgeB, 0, qbA, kbA, vbA, msgA)
        issue_scatter(msgA, dsA, semSA)
        wait_gathers(daB, saB, qbB, kbB, vbB, semB)
        wait_scatter(msgB, dsB, semSB)
        copy_ds(bgdB, _CB, dsB)
        compute(_CB, bgeB, _CB, qbB, kbB, vbB, msgB)
        issue_scatter(msgB, dsB, semSB)
        load_idx_pair(2 * i + 3, bgdB, bgsB, bgeB, semIB)
        wait_idx(bgdA, bgsA, bgeA, semIA)
        adjust(bgdA, bgsA, 0, daA, saA)
        issue_gathers(daA, saA, qbA, kbA, vbA, semA)
        return 0
    lax.fori_loop(0, (_NCH - 4) // 4, quad, 0)

    adjust(bgdA, bgsA, _CB, daB, saB)
    issue_gathers(daB, saB, qbB, kbB, vbB, semB)
    wait_gathers(daA, saA, qbA, kbA, vbA, semA)
    wait_scatter(msgA, dsA, semSA)
    copy_ds(bgdA, 0, dsA)
    compute(_CB, bgeA, 0, qbA, kbA, vbA, msgA)
    issue_scatter(msgA, dsA, semSA)
    wait_idx(bgdB, bgsB, bgeB, semIB)
    adjust(bgdB, bgsB, 0, daA, saA)
    issue_gathers(daA, saA, qbA, kbA, vbA, semA)
    wait_gathers(daB, saB, qbB, kbB, vbB, semB)
    wait_scatter(msgB, dsB, semSB)
    copy_ds(bgdA, _CB, dsB)
    compute(_CB, bgeA, _CB, qbB, kbB, vbB, msgB)
    issue_scatter(msgB, dsB, semSB)
    adjust(bgdB, bgsB, _CB, daB, saB)
    issue_gathers(daB, saB, qbB, kbB, vbB, semB)
    wait_gathers(daA, saA, qbA, kbA, vbA, semA)
    wait_scatter(msgA, dsA, semSA)
    copy_ds(bgdB, 0, dsA)
    compute(_CB, bgeB, 0, qbA, kbA, vbA, msgA)
    issue_scatter(msgA, dsA, semSA)
    wait_gathers(daB, saB, qbB, kbB, vbB, semB)
    wait_scatter(msgB, dsB, semSB)
    copy_ds(bgdB, _CB, dsB)
    compute(_CB, bgeB, _CB, qbB, kbB, vbB, msgB)
    issue_scatter(msgB, dsB, semSB)

    rbase = ebase + _NCH * _CB
    pltpu.sync_copy(dst1.at[pl.ds(rbase, _ER)], dc2)
    pltpu.sync_copy(src1.at[pl.ds(rbase, _ER)], sa2)
    pltpu.sync_copy(et1.at[pl.ds(rbase, _ER)], bgeA.at[pl.ds(0, _ER)])
    da2[...] = dc2[...] + coff
    sa2[...] = sa2[...] + coff
    r1 = pltpu.async_copy(q2.at[da2], qbA.at[pl.ds(0, _ER)], semA)
    r2 = pltpu.async_copy(k2.at[sa2], kbA.at[pl.ds(0, _ER)], semA)
    r3 = pltpu.async_copy(v2.at[sa2], vbA.at[pl.ds(0, _ER)], semA)
    r1.wait()
    r2.wait()
    r3.wait()
    wait_scatter(msgA, dsA, semSA)
    compute(_ER, bgeA, 0, qbA, kbA, vbA, msgA)
    pltpu.sync_copy(msgA.at[pl.ds(0, _ER)], acc.at[dc2], add=True)
    wait_scatter(msgB, dsB, semSB)

    plsc.subcore_barrier()

    def nblock(rbase):
        pltpu.sync_copy(acc.at[pl.ds(rbase, _RB)], msgA.at[pl.ds(0, _RB)])

        def nrow(r, _):
            inv = 1.0 / (msgA[r, pl.ds(128, 16)] + 1e-16)
            rv = lax.rev(inv, dimensions=(0,))
            inv0 = jnp.where(lo, inv, rv)
            inv1 = jnp.where(lo, rv, inv)
            for j in range(8):
                sl = pl.ds(j * 16, 16)
                ob[r, sl] = msgA[r, sl] * (inv0 if j < 4 else inv1)
            return 0
        lax.fori_loop(0, _RB, nrow, 0)
        pltpu.sync_copy(ob, out2.at[pl.ds(c * _N + rbase, _RB)])

    def nblk(b, _):
        nblock(s * _NPS + b * _RB)
        return 0
    lax.fori_loop(0, _NPS // _RB, nblk, 0)

    @pl.when(s == 0)
    def _():
        nblock(_NS * _NPS)
        nblock(_NS * _NPS + _RB)


def _edge(q2, k2, v2, rw, dst1, src1, et1):
    f32, i32 = jnp.float32, jnp.int32
    mesh = plsc.VectorSubcoreMesh(core_axis_name="c", subcore_axis_name="s",
                                  num_cores=2, num_subcores=_NS)
    return pl.kernel(
        _edge_body,
        out_type=jax.ShapeDtypeStruct((2 * _N, _CH), f32),
        mesh=mesh,
        compiler_params=pltpu.CompilerParams(use_tc_tiling_on_sc=False,
                                             needs_layout_passes=False),
        scratch_types=[
            pltpu.VMEM_SHARED((_N, _ACCW), f32),
            pltpu.VMEM((24, _CH), f32),
            pltpu.VMEM((_CB, _CH), f32),
            pltpu.VMEM((_CB, _CH), f32),
            pltpu.VMEM((_CB, _CH), f32),
            pltpu.VMEM((_CB, _CH), f32),
            pltpu.VMEM((_CB, _CH), f32),
            pltpu.VMEM((_CB, _CH), f32),
            pltpu.VMEM((_CB, _ACCW), f32),
            pltpu.VMEM((_CB, _ACCW), f32),
            pltpu.VMEM((2 * _CB,), i32),
            pltpu.VMEM((2 * _CB,), i32),
            pltpu.VMEM((2 * _CB + 16,), i32),
            pltpu.VMEM((2 * _CB,), i32),
            pltpu.VMEM((2 * _CB,), i32),
            pltpu.VMEM((2 * _CB + 16,), i32),
            pltpu.VMEM((_CB,), i32),
            pltpu.VMEM((_CB,), i32),
            pltpu.VMEM((_CB,), i32),
            pltpu.VMEM((_CB,), i32),
            pltpu.VMEM((_CB,), i32),
            pltpu.VMEM((_CB,), i32),
            pltpu.VMEM((_ER,), i32),
            pltpu.VMEM((_ER,), i32),
            pltpu.VMEM((_ER,), i32),
            pltpu.VMEM((_RB, _CH), f32),
            pltpu.SemaphoreType.DMA,
            pltpu.SemaphoreType.DMA,
            pltpu.SemaphoreType.DMA,
            pltpu.SemaphoreType.DMA,
            pltpu.SemaphoreType.DMA,
            pltpu.SemaphoreType.DMA,
        ],
    )(q2, k2, v2, rw, dst1, src1, et1)


def _post_body(olo_ref, ohi_ref, xr_ref, x_ref, wb_ref, g2_ref, b2_ref,
               wf1_ref, bf1_ref, wf2_ref, bf2_ref, o_ref):
    out = jnp.concatenate([olo_ref[...], ohi_ref[...]], axis=1)
    xr = xr_ref[...]
    wb = wb_ref[...]
    gp = (jnp.sum(out * (wb[0:1, :] + wb[2:3, :]), axis=1, keepdims=True)
          + jnp.sum(xr * (wb[1:2, :] - wb[2:3, :]), axis=1, keepdims=True))
    g = jax.nn.sigmoid(gp)
    h = x_ref[...] + g * xr + (1.0 - g) * out
    mu = jnp.mean(h, axis=1, keepdims=True)
    hc = h - mu
    var = jnp.mean(hc * hc, axis=1, keepdims=True)
    hn = hc * lax.rsqrt(var + 1e-5) * g2_ref[...] + b2_ref[...]
    t = jnp.dot(hn, wf1_ref[...], preferred_element_type=jnp.float32) + bf1_ref[...]
    t = 0.5 * t * (1.0 + lax.erf(t * 0.7071067811865476))
    hf = jnp.dot(t, wf2_ref[...], preferred_element_type=jnp.float32) + bf2_ref[...]
    o_ref[...] = h + hf


def _post(olo, ohi, xr, x, Wb3, ln2_g, ln2_b, Wf1, bf1, Wf2, bf2):
    row_h = pl.BlockSpec((_BN, _CH), lambda i: (i, 0))
    row = pl.BlockSpec((_BN, _D), lambda i: (i, 0))

    def full(a, b):
        return pl.BlockSpec((a, b), lambda i: (0, 0))

    return pl.pallas_call(
        _post_body,
        grid=(_GI,),
        in_specs=[row_h, row_h, row, row, full(3, _D), full(1, _D), full(1, _D),
                  full(_D, 4 * _D), full(1, 4 * _D), full(4 * _D, _D), full(1, _D)],
        out_specs=row,
        out_shape=jax.ShapeDtypeStruct((_N, _D), jnp.float32),
    )(olo, ohi, xr, x, Wb3, ln2_g.reshape(1, _D), ln2_b.reshape(1, _D),
      Wf1, bf1.reshape(1, 4 * _D), Wf2, bf2.reshape(1, _D))


def kernel(x, edge_index, edge_type, rel_emb, Wq, bq, Wk, bk, Wv, bv, We, be,
           Wskip, bskip, Wbeta, ln1_g, ln1_b, ln2_g, ln2_b, Wf1, bf1, Wf2, bf2):
    q2, k2, v2, xr = _pre(x, ln1_g, ln1_b, Wq, bq, Wk, bk, Wv, bv, Wskip, bskip)
    rel_pad = jnp.zeros((24, 64), jnp.float32).at[:17].set(rel_emb)
    rw = _relwe(rel_pad, We, be)
    out2 = _edge(q2, k2, v2, rw, edge_index[1], edge_index[0], edge_type)
    return _post(out2[:_N], out2[_N:], xr, x, Wbeta.reshape(3, _D),
                 ln2_g, ln2_b, Wf1, bf1, Wf2, bf2)


# speedup vs baseline: 1.1847x; 1.0582x over previous
import jax
import jax.numpy as jnp
from jax import lax
from jax.experimental import pallas as pl
from jax.experimental.pallas import tpu as pltpu
from jax.experimental.pallas import tpu_sc as plsc

_N = 10000
_E = 160000
_D = 256
_HC = 256
_CH = 128
_NS = 16
_CB = 32
_EPS = _E // _NS
_NCH = _EPS // _CB
_ER = _EPS - _NCH * _CB
_NPS = 624
_RB = 8
_ACCW = 144
_BN = 1000
_GI = _N // _BN


def _pre_body(x_ref, g_ref, b_ref, wq_ref, bq_ref, wk_ref, bk_ref,
              wv_ref, bv_ref, ws_ref, bs_ref,
              q_ref, k_ref, v_ref, xr_ref):
    xb = x_ref[...]
    mu = jnp.mean(xb, axis=1, keepdims=True)
    xc = xb - mu
    var = jnp.mean(xc * xc, axis=1, keepdims=True)
    xn = xc * lax.rsqrt(var + 1e-5) * g_ref[...] + b_ref[...]
    q_ref[...] = jnp.dot(xn, wq_ref[...], preferred_element_type=jnp.float32) + bq_ref[...]
    k_ref[...] = jnp.dot(xn, wk_ref[...], preferred_element_type=jnp.float32) + bk_ref[...]
    v_ref[...] = jnp.dot(xn, wv_ref[...], preferred_element_type=jnp.float32) + bv_ref[...]
    xr_ref[...] = jnp.dot(xn, ws_ref[...], preferred_element_type=jnp.float32) + bs_ref[...]


def _pre(x, ln1_g, ln1_b, Wq, bq, Wk, bk, Wv, bv, Wskip, bskip):
    row = pl.BlockSpec((_BN, _D), lambda i, c: (i, 0))
    wcol = pl.BlockSpec((_D, _CH), lambda i, c: (0, c))
    bcol = pl.BlockSpec((1, _CH), lambda i, c: (0, c))
    vec = pl.BlockSpec((1, _D), lambda i, c: (0, 0))
    q_out = pl.BlockSpec((_BN, _CH), lambda i, c: (c * _GI + i, 0))
    f32 = jnp.float32
    return pl.pallas_call(
        _pre_body,
        grid=(_GI, 2),
        in_specs=[row, vec, vec, wcol, bcol, wcol, bcol, wcol, bcol, wcol, bcol],
        out_specs=[q_out, q_out, q_out,
                   pl.BlockSpec((_BN, _CH), lambda i, c: (i, c))],
        out_shape=[jax.ShapeDtypeStruct((2 * _N, _CH), f32)] * 3
        + [jax.ShapeDtypeStruct((_N, _D), f32)],
    )(x, ln1_g.reshape(1, _D), ln1_b.reshape(1, _D), Wq, bq.reshape(1, _HC),
      Wk, bk.reshape(1, _HC), Wv, bv.reshape(1, _HC), Wskip, bskip.reshape(1, _HC))


def _relwe_body(rel_ref, we_ref, be_ref, out_ref):
    out_ref[...] = jnp.dot(rel_ref[...], we_ref[...],
                           preferred_element_type=jnp.float32) + be_ref[...]


def _relwe(rel_pad, We, be):
    return pl.pallas_call(
        _relwe_body,
        grid=(2,),
        in_specs=[pl.BlockSpec((24, 64), lambda c: (0, 0)),
                  pl.BlockSpec((64, _CH), lambda c: (0, c)),
                  pl.BlockSpec((1, _CH), lambda c: (0, c))],
        out_specs=pl.BlockSpec((24, _CH), lambda c: (c, 0)),
        out_shape=jax.ShapeDtypeStruct((48, _CH), jnp.float32),
    )(rel_pad, We, be.reshape(1, _HC))


def _edge_body(q2, k2, v2, rw, dst1, src1, et1, out2,
               acc, rwb,
               qbA, kbA, vbA, qbB, kbB, vbB, msgA, msgB,
               bgdA, bgsA, bgeA, bgdB, bgsB, bgeB,
               daA, saA, daB, saB, dsA, dsB,
               dc2, da2, sa2,
               ob, semA, semB, semIA, semIB, semSA, semSB):
    c = lax.axis_index("c")
    s = lax.axis_index("s")
    coff = jnp.full((16,), c * _N, dtype=jnp.int32)
    iota = lax.iota(jnp.int32, 16)
    lo = iota < 8
    px8 = iota ^ 8
    px4 = iota ^ 4
    px2 = iota ^ 2
    px1 = iota ^ 1
    inv_sqrt_c = 0.125

    _dn = lax.GatherDimensionNumbers(offset_dims=(), collapsed_slice_dims=(0,),
                                     start_index_map=(0,))

    def shuf(x, idx):
        return lax.gather(x, idx[:, None], _dn, slice_sizes=(1,),
                          mode=lax.GatherScatterMode.PROMISE_IN_BOUNDS)

    pltpu.sync_copy(rw.at[pl.ds(c * 24, 24)], rwb)

    def zrow(r, _):
        for j in range(_ACCW // 16):
            msgA[r, pl.ds(j * 16, 16)] = jnp.zeros((16,), jnp.float32)
        return 0
    lax.fori_loop(0, _RB, zrow, 0)

    def zblk(b, _):
        pltpu.sync_copy(msgA.at[pl.ds(0, _RB)],
                        acc.at[pl.ds(s * _NPS + b * _RB, _RB)])
        return 0
    lax.fori_loop(0, _NPS // _RB, zblk, 0)

    @pl.when(s == 0)
    def _():
        pltpu.sync_copy(msgA.at[pl.ds(0, _RB)], acc.at[pl.ds(_NS * _NPS, _RB)])
        pltpu.sync_copy(msgA.at[pl.ds(0, _RB)],
                        acc.at[pl.ds(_NS * _NPS + _RB, _RB)])

    plsc.subcore_barrier()

    ebase = s * _EPS

    def load_idx_pair(pair, bgd, bgs, bge, sem):
        b0 = ebase + pair * 2 * _CB
        pltpu.async_copy(dst1.at[pl.ds(b0, 2 * _CB)], bgd, sem)
        pltpu.async_copy(src1.at[pl.ds(b0, 2 * _CB)], bgs, sem)
        pltpu.async_copy(et1.at[pl.ds(b0, 2 * _CB)], bge.at[pl.ds(0, 2 * _CB)], sem)

    def wait_idx(bgd, bgs, bge, sem):
        pltpu.make_async_copy(dst1.at[pl.ds(0, 2 * _CB)], bgd, sem).wait()
        pltpu.make_async_copy(src1.at[pl.ds(0, 2 * _CB)], bgs, sem).wait()
        pltpu.make_async_copy(et1.at[pl.ds(0, 2 * _CB)],
                              bge.at[pl.ds(0, 2 * _CB)], sem).wait()

    def adjust(bgd, bgs, off, da, sa):
        for t in range(_CB // 16):
            sl = pl.ds(t * 16, 16)
            so = pl.ds(off + t * 16, 16)
            da[sl] = bgd[so] + coff
            sa[sl] = bgs[so] + coff

    def issue_gathers(da, sa, qb, kb, vb, sem):
        pltpu.async_copy(q2.at[da], qb, sem)
        pltpu.async_copy(k2.at[sa], kb, sem)
        pltpu.async_copy(v2.at[sa], vb, sem)

    def wait_gathers(da, sa, qb, kb, vb, sem):
        pltpu.make_async_copy(q2.at[da], qb, sem).wait()
        pltpu.make_async_copy(k2.at[sa], kb, sem).wait()
        pltpu.make_async_copy(v2.at[sa], vb, sem).wait()

    def wait_scatter(msg, ds, sem):
        pltpu.make_async_copy(msg, acc.at[ds], sem).wait()

    def compute(nedge, bge, eoff, qb, kb, vb, msg):
        @plsc.parallel_loop(0, nedge, 1, unroll=4)
        def edge(e):
            t = bge[pl.ds(eoff + e, 16)][0]
            a0 = jnp.zeros((16,), jnp.float32)
            a1 = jnp.zeros((16,), jnp.float32)
            vs = []
            for j in range(8):
                sl = pl.ds(j * 16, 16)
                ee = rwb[t, sl]
                kj = kb[e, sl] + ee
                vj = vb[e, sl] + ee
                qj = qb[e, sl]
                if j < 4:
                    a0 = a0 + qj * kj
                else:
                    a1 = a1 + qj * kj
                vs.append(vj)
            m = jnp.where(lo, a0 + shuf(a0, px8), a1 + shuf(a1, px8))
            m = m + shuf(m, px4)
            m = m + shuf(m, px2)
            m = m + shuf(m, px1)
            exd = jnp.exp(m * inv_sqrt_c)
            rxd = lax.rev(exd, dimensions=(0,))
            ex0 = jnp.where(lo, exd, rxd)
            ex1 = jnp.where(lo, rxd, exd)
            for j in range(8):
                msg[e, pl.ds(j * 16, 16)] = vs[j] * (ex0 if j < 4 else ex1)
            msg[e, pl.ds(128, 16)] = exd

    def copy_ds(bgd, off, ds):
        for t in range(_CB // 16):
            ds[pl.ds(t * 16, 16)] = bgd[pl.ds(off + t * 16, 16)]

    def issue_scatter(msg, ds, sem):
        pltpu.async_copy(msg, acc.at[ds], sem, add=True)

    pltpu.sync_copy(dst1.at[pl.ds(ebase, 2 * _CB)], bgdA)
    pltpu.sync_copy(src1.at[pl.ds(ebase, 2 * _CB)], bgsA)
    pltpu.sync_copy(et1.at[pl.ds(ebase, 2 * _CB)], bgeA.at[pl.ds(0, 2 * _CB)])
    load_idx_pair(1, bgdB, bgsB, bgeB, semIB)
    adjust(bgdA, bgsA, 0, daA, saA)
    issue_gathers(daA, saA, qbA, kbA, vbA, semA)

    def quad(i, _):
        adjust(bgdA, bgsA, _CB, daB, saB)
        issue_gathers(daB, saB, qbB, kbB, vbB, semB)
        wait_gathers(daA, saA, qbA, kbA, vbA, semA)

        @pl.when(i > 0)
        def _():
            wait_scatter(msgA, dsA, semSA)
        copy_ds(bgdA, 0, dsA)
        compute(_CB, bgeA, 0, qbA, kbA, vbA, msgA)
        issue_scatter(msgA, dsA, semSA)
        wait_idx(bgdB, bgsB, bgeB, semIB)
        adjust(bgdB, bgsB, 0, daA, saA)
        issue_gathers(daA, saA, qbA, kbA, vbA, semA)
        wait_gathers(daB, saB, qbB, kbB, vbB, semB)

        @pl.when(i > 0)
        def _():
            wait_scatter(msgB, dsB, semSB)
        copy_ds(bgdA, _CB, dsB)
        compute(_CB, bgeA, _CB, qbB, kbB, vbB, msgB)
        issue_scatter(msgB, dsB, semSB)
        load_idx_pair(2 * i + 2, bgdA, bgsA, bgeA, semIA)
        adjust(bgdB, bgsB, _CB, daB, saB)
        issue_gathers(daB, saB, qbB, kbB, vbB, semB)
        wait_gathers(daA, saA, qbA, kbA, vbA, semA)
        wait_scatter(msgA, dsA, semSA)
        copy_ds(bgdB, 0, dsA)
        compute(_CB, bgeB, 0, qbA, kbA, vbA, msgA)
        issue_scatter(msgA, dsA, semSA)
        wait_gathers(daB, saB, qbB, kbB, vbB, semB)
        wait_scatter(msgB, dsB, semSB)
        copy_ds(bgdB, _CB, dsB)
        compute(_CB, bgeB, _CB, qbB, kbB, vbB, msgB)
        issue_scatter(msgB, dsB, semSB)
        load_idx_pair(2 * i + 3, bgdB, bgsB, bgeB, semIB)
        wait_idx(bgdA, bgsA, bgeA, semIA)
        adjust(bgdA, bgsA, 0, daA, saA)
        issue_gathers(daA, saA, qbA, kbA, vbA, semA)
        return 0
    lax.fori_loop(0, (_NCH - 4) // 4, quad, 0)

    adjust(bgdA, bgsA, _CB, daB, saB)
    issue_gathers(daB, saB, qbB, kbB, vbB, semB)
    wait_gathers(daA, saA, qbA, kbA, vbA, semA)
    wait_scatter(msgA, dsA, semSA)
    copy_ds(bgdA, 0, dsA)
    compute(_CB, bgeA, 0, qbA, kbA, vbA, msgA)
    issue_scatter(msgA, dsA, semSA)
    wait_idx(bgdB, bgsB, bgeB, semIB)
    adjust(bgdB, bgsB, 0, daA, saA)
    issue_gathers(daA, saA, qbA, kbA, vbA, semA)
    wait_gathers(daB, saB, qbB, kbB, vbB, semB)
    wait_scatter(msgB, dsB, semSB)
    copy_ds(bgdA, _CB, dsB)
    compute(_CB, bgeA, _CB, qbB, kbB, vbB, msgB)
    issue_scatter(msgB, dsB, semSB)
    adjust(bgdB, bgsB, _CB, daB, saB)
    issue_gathers(daB, saB, qbB, kbB, vbB, semB)
    wait_gathers(daA, saA, qbA, kbA, vbA, semA)
    wait_scatter(msgA, dsA, semSA)
    copy_ds(bgdB, 0, dsA)
    compute(_CB, bgeB, 0, qbA, kbA, vbA, msgA)
    issue_scatter(msgA, dsA, semSA)
    wait_gathers(daB, saB, qbB, kbB, vbB, semB)
    wait_scatter(msgB, dsB, semSB)
    copy_ds(bgdB, _CB, dsB)
    compute(_CB, bgeB, _CB, qbB, kbB, vbB, msgB)
    issue_scatter(msgB, dsB, semSB)

    rbase = ebase + _NCH * _CB
    pltpu.sync_copy(dst1.at[pl.ds(rbase, _ER)], dc2)
    pltpu.sync_copy(src1.at[pl.ds(rbase, _ER)], sa2)
    pltpu.sync_copy(et1.at[pl.ds(rbase, _ER)], bgeA.at[pl.ds(0, _ER)])
    da2[...] = dc2[...] + coff
    sa2[...] = sa2[...] + coff
    r1 = pltpu.async_copy(q2.at[da2], qbA.at[pl.ds(0, _ER)], semA)
    r2 = pltpu.async_copy(k2.at[sa2], kbA.at[pl.ds(0, _ER)], semA)
    r3 = pltpu.async_copy(v2.at[sa2], vbA.at[pl.ds(0, _ER)], semA)
    r1.wait()
    r2.wait()
    r3.wait()
    wait_scatter(msgA, dsA, semSA)
    compute(_ER, bgeA, 0, qbA, kbA, vbA, msgA)
    pltpu.sync_copy(msgA.at[pl.ds(0, _ER)], acc.at[dc2], add=True)
    wait_scatter(msgB, dsB, semSB)

    plsc.subcore_barrier()

    def nblock(rbase):
        pltpu.sync_copy(acc.at[pl.ds(rbase, _RB)], msgA.at[pl.ds(0, _RB)])

        def nrow(r, _):
            inv = 1.0 / (msgA[r, pl.ds(128, 16)] + 1e-16)
            rv = lax.rev(inv, dimensions=(0,))
            inv0 = jnp.where(lo, inv, rv)
            inv1 = jnp.where(lo, rv, inv)
            for j in range(8):
                sl = pl.ds(j * 16, 16)
                ob[r, sl] = msgA[r, sl] * (inv0 if j < 4 else inv1)
            return 0
        lax.fori_loop(0, _RB, nrow, 0)
        pltpu.sync_copy(ob, out2.at[pl.ds(c * _N + rbase, _RB)])

    def nblk(b, _):
        nblock(s * _NPS + b * _RB)
        return 0
    lax.fori_loop(0, _NPS // _RB, nblk, 0)

    @pl.when(s == 0)
    def _():
        nblock(_NS * _NPS)
        nblock(_NS * _NPS + _RB)


def _edge(q2, k2, v2, rw, dst1, src1, et1):
    f32, i32 = jnp.float32, jnp.int32
    mesh = plsc.VectorSubcoreMesh(core_axis_name="c", subcore_axis_name="s",
                                  num_cores=2, num_subcores=_NS)
    return pl.kernel(
        _edge_body,
        out_type=jax.ShapeDtypeStruct((2 * _N, _CH), f32),
        mesh=mesh,
        compiler_params=pltpu.CompilerParams(use_tc_tiling_on_sc=False,
                                             needs_layout_passes=False),
        scratch_types=[
            pltpu.VMEM_SHARED((_N, _ACCW), f32),
            pltpu.VMEM((24, _CH), f32),
            pltpu.VMEM((_CB, _CH), f32),
            pltpu.VMEM((_CB, _CH), f32),
            pltpu.VMEM((_CB, _CH), f32),
            pltpu.VMEM((_CB, _CH), f32),
            pltpu.VMEM((_CB, _CH), f32),
            pltpu.VMEM((_CB, _CH), f32),
            pltpu.VMEM((_CB, _ACCW), f32),
            pltpu.VMEM((_CB, _ACCW), f32),
            pltpu.VMEM((2 * _CB,), i32),
            pltpu.VMEM((2 * _CB,), i32),
            pltpu.VMEM((2 * _CB + 16,), i32),
            pltpu.VMEM((2 * _CB,), i32),
            pltpu.VMEM((2 * _CB,), i32),
            pltpu.VMEM((2 * _CB + 16,), i32),
            pltpu.VMEM((_CB,), i32),
            pltpu.VMEM((_CB,), i32),
            pltpu.VMEM((_CB,), i32),
            pltpu.VMEM((_CB,), i32),
            pltpu.VMEM((_CB,), i32),
            pltpu.VMEM((_CB,), i32),
            pltpu.VMEM((_ER,), i32),
            pltpu.VMEM((_ER,), i32),
            pltpu.VMEM((_ER,), i32),
            pltpu.VMEM((_RB, _CH), f32),
            pltpu.SemaphoreType.DMA,
            pltpu.SemaphoreType.DMA,
            pltpu.SemaphoreType.DMA,
            pltpu.SemaphoreType.DMA,
            pltpu.SemaphoreType.DMA,
            pltpu.SemaphoreType.DMA,
        ],
    )(q2, k2, v2, rw, dst1, src1, et1)


def _post_body(olo_ref, ohi_ref, xr_ref, x_ref, wb_ref, g2_ref, b2_ref,
               wf1_ref, bf1_ref, wf2_ref, bf2_ref, o_ref):
    out = jnp.concatenate([olo_ref[...], ohi_ref[...]], axis=1)
    xr = xr_ref[...]
    wb = wb_ref[...]
    gp = (jnp.sum(out * (wb[0:1, :] + wb[2:3, :]), axis=1, keepdims=True)
          + jnp.sum(xr * (wb[1:2, :] - wb[2:3, :]), axis=1, keepdims=True))
    g = jax.nn.sigmoid(gp)
    h = x_ref[...] + g * xr + (1.0 - g) * out
    mu = jnp.mean(h, axis=1, keepdims=True)
    hc = h - mu
    var = jnp.mean(hc * hc, axis=1, keepdims=True)
    hn = hc * lax.rsqrt(var + 1e-5) * g2_ref[...] + b2_ref[...]
    t = jnp.dot(hn, wf1_ref[...], preferred_element_type=jnp.float32) + bf1_ref[...]
    t = 0.5 * t * (1.0 + lax.erf(t * 0.7071067811865476))
    hf = jnp.dot(t, wf2_ref[...], preferred_element_type=jnp.float32) + bf2_ref[...]
    o_ref[...] = h + hf


def _post(olo, ohi, xr, x, Wb3, ln2_g, ln2_b, Wf1, bf1, Wf2, bf2):
    row_h = pl.BlockSpec((_BN, _CH), lambda i: (i, 0))
    row = pl.BlockSpec((_BN, _D), lambda i: (i, 0))

    def full(a, b):
        return pl.BlockSpec((a, b), lambda i: (0, 0))

    return pl.pallas_call(
        _post_body,
        grid=(_GI,),
        in_specs=[row_h, row_h, row, row, full(3, _D), full(1, _D), full(1, _D),
                  full(_D, 4 * _D), full(1, 4 * _D), full(4 * _D, _D), full(1, _D)],
        out_specs=row,
        out_shape=jax.ShapeDtypeStruct((_N, _D), jnp.float32),
    )(olo, ohi, xr, x, Wb3, ln2_g.reshape(1, _D), ln2_b.reshape(1, _D),
      Wf1, bf1.reshape(1, 4 * _D), Wf2, bf2.reshape(1, _D))


def kernel(x, edge_index, edge_type, rel_emb, Wq, bq, Wk, bk, Wv, bv, We, be,
           Wskip, bskip, Wbeta, ln1_g, ln1_b, ln2_g, ln2_b, Wf1, bf1, Wf2, bf2):
    q2, k2, v2, xr = _pre(x, ln1_g, ln1_b, Wq, bq, Wk, bk, Wv, bv, Wskip, bskip)
    rel_pad = jnp.zeros((24, 64), jnp.float32).at[:17].set(rel_emb)
    rw = _relwe(rel_pad, We, be)
    out2 = _edge(q2, k2, v2, rw, edge_index[1], edge_index[0], edge_type)
    return _post(out2[:_N], out2[_N:], xr, x, Wbeta.reshape(3, _D),
                 ln2_g, ln2_b, Wf1, bf1, Wf2, bf2)
